# baseline (speedup 1.0000x reference)
import jax
import jax.numpy as jnp
from jax.experimental import pallas as pl
from jax.experimental.pallas import tpu as pltpu

_BETA = 0.25
_N_TOK = 2048
_CODE_DIM = 256
_K = 1024
_BLK = 256
_NB = _N_TOK // _BLK


def _vq_kernel(z_hbm, c_hbm, zq_hbm, part_ref, z_s, c_s, zq_s, zsem, csem, osem):
    ccopy = pltpu.make_async_copy(c_hbm, c_s, csem)
    ccopy.start()
    zcopies = []
    for h in range(_NB):
        cp = pltpu.make_async_copy(
            z_hbm.at[pl.ds(h * _BLK, _BLK), :],
            z_s.at[pl.ds(h * _BLK, _BLK), :], zsem.at[h])
        cp.start()
        zcopies.append(cp)
    ccopy.wait()
    c = c_s[...]
    c2 = jnp.sum(c * c, axis=1)[None, :]
    cbf = c.astype(jnp.bfloat16)
    total = jnp.zeros((), jnp.float32)
    ocopies = []
    for h in range(_NB):
        zcopies[h].wait()
        z = z_s[pl.ds(h * _BLK, _BLK), :]
        m = jnp.dot(z, c.T, preferred_element_type=jnp.float32)
        z2 = jnp.sum(z * z, axis=1, keepdims=True)
        dist = z2 - 2.0 * m + c2
        rowmin = jnp.min(dist, axis=1, keepdims=True)
        iota = jax.lax.broadcasted_iota(jnp.int32, dist.shape, 1).astype(jnp.float32)
        idx = jnp.min(jnp.where(dist == rowmin, iota, float(_K)), axis=1,
                      keepdims=True)
        onehot = (iota == idx).astype(jnp.bfloat16)
        zq = jnp.dot(onehot, cbf, preferred_element_type=jnp.float32)
        zq_s[pl.ds(h * _BLK, _BLK), :] = zq
        ocp = pltpu.make_async_copy(
            zq_s.at[pl.ds(h * _BLK, _BLK), :],
            zq_hbm.at[pl.ds(h * _BLK, _BLK), :], osem.at[h])
        ocp.start()
        ocopies.append(ocp)
        total = total + jnp.sum(rowmin)
    part_ref[...] = jnp.full((1, 128), total, jnp.float32)
    for ocp in ocopies:
        ocp.wait()


def kernel(z, codebook):
    z = z.reshape(z.shape[0], -1)
    zq, parts = pl.pallas_call(
        _vq_kernel,
        in_specs=[
            pl.BlockSpec(memory_space=pltpu.MemorySpace.HBM),
            pl.BlockSpec(memory_space=pltpu.MemorySpace.HBM),
        ],
        out_specs=[
            pl.BlockSpec(memory_space=pltpu.MemorySpace.HBM),
            pl.BlockSpec(memory_space=pltpu.MemorySpace.VMEM),
        ],
        out_shape=[
            jax.ShapeDtypeStruct((_N_TOK, _CODE_DIM), jnp.float32),
            jax.ShapeDtypeStruct((1, 128), jnp.float32),
        ],
        scratch_shapes=[
            pltpu.VMEM((_N_TOK, _CODE_DIM), jnp.float32),
            pltpu.VMEM((_K, _CODE_DIM), jnp.float32),
            pltpu.VMEM((_N_TOK, _CODE_DIM), jnp.float32),
            pltpu.SemaphoreType.DMA((_NB,)),
            pltpu.SemaphoreType.DMA,
            pltpu.SemaphoreType.DMA((_NB,)),
        ],
    )(z, codebook)
    mean_sq = parts[0, 0] / (_N_TOK * _CODE_DIM)
    loss = _BETA * mean_sq + mean_sq
    return (zq, loss)
